# baseline (speedup 1.0000x reference)
import functools

import jax
import jax.numpy as jnp
from jax import lax
from jax.experimental import pallas as pl
from jax.experimental.pallas import tpu as pltpu
from jax.experimental.pallas import tpu_sc as plsc

VOCAB = 100000
HID = 128
CTX = 200
NROW = 1024 * 200
NC = 2
NS = 16
NW = NC * NS
RPW = NROW // NW
CHUNK = 128
NCHUNK = RPW // CHUNK
LANES = 16
NGRP = CHUNK // LANES


def _combine_body(pos_ref, tok_ref, out_ref):
    out_ref[0:CTX, :] = pos_ref[...] + tok_ref[0:1, :]
    out_ref[CTX:2 * CTX, :] = pos_ref[...] + tok_ref[1:2, :]


def _build_combined(pos_emb, tok_emb):
    return pl.pallas_call(
        _combine_body,
        out_shape=jax.ShapeDtypeStruct((2 * CTX, HID), jnp.float32),
    )(pos_emb, tok_emb)


_sc_mesh = plsc.VectorSubcoreMesh(core_axis_name="c", subcore_axis_name="s")


@functools.partial(
    pl.kernel,
    out_type=jax.ShapeDtypeStruct((NROW, HID), jnp.float32),
    mesh=_sc_mesh,
    scratch_types=[
        pltpu.VMEM((2 * CTX * HID,), jnp.float32),
        pltpu.VMEM((CHUNK,), jnp.int32),
        pltpu.VMEM((CHUNK,), jnp.int32),
        pltpu.VMEM((CHUNK,), jnp.int32),
        pltpu.VMEM((CHUNK,), jnp.int32),
        pltpu.VMEM((CHUNK,), jnp.int32),
        pltpu.VMEM((CHUNK,), jnp.int32),
        pltpu.VMEM((CHUNK, HID), jnp.float32),
        pltpu.VMEM((CHUNK, HID), jnp.float32),
        pltpu.SemaphoreType.DMA,
        pltpu.SemaphoreType.DMA,
        pltpu.SemaphoreType.DMA,
    ],
    compiler_params=pltpu.CompilerParams(needs_layout_passes=False),
)
def _sc_embed(word_hbm, comb_hbm, ids_hbm, tt_hbm, out_hbm,
              comb_v, widx0, widx1, ttv0, ttv1, cb0, cb1, wrows0, wrows1,
              csem, gsem0, gsem1):
    wid = lax.axis_index("s") * NC + lax.axis_index("c")
    row0 = wid * RPW
    widxs = (widx0, widx1)
    ttvs = (ttv0, ttv1)
    cbs = (cb0, cb1)
    wrowss = (wrows0, wrows1)
    gsems = (gsem0, gsem1)

    comb_cp = pltpu.async_copy(comb_hbm, comb_v, csem)

    def prep_idx(ci, b):
        base = row0 + ci * CHUNK
        pltpu.sync_copy(ids_hbm.at[pl.ds(base, CHUNK)], widxs[b])
        pltpu.sync_copy(tt_hbm.at[pl.ds(base, CHUNK)], ttvs[b])

        def idx_body(j, _):
            o = j * LANES
            n = base + o + lax.iota(jnp.int32, LANES)
            s = n % CTX
            cbs[b][pl.ds(o, LANES)] = (ttvs[b][pl.ds(o, LANES)] * CTX + s) * HID
            return 0

        lax.fori_loop(0, CHUNK // LANES, idx_body, 0)

    def start_gather(b):
        pltpu.async_copy(word_hbm.at[widxs[b]], wrowss[b], gsems[b])

    def wait_gather(b):
        pltpu.make_async_copy(word_hbm.at[widxs[b]], wrowss[b],
                              gsems[b]).wait()

    def finish_chunk(ci, b):
        base = row0 + ci * CHUNK
        wait_gather(b)
        cvecs = []
        rvecs = []
        for g in range(NGRP):
            o = g * LANES
            cvecs.append(cbs[b][pl.ds(o, LANES)])
            rvecs.append(o + lax.iota(jnp.int32, LANES))

        def col_body(c, carry):
            cv, rv = carry
            csplat = jnp.zeros((LANES,), jnp.int32) + c
            for g in range(NGRP):
                vals = plsc.load_gather(comb_v, [cv[g] + c])
                plsc.addupdate_scatter(wrowss[b], [rv[g], csplat], vals)
            return carry

        lax.fori_loop(0, HID, col_body, (tuple(cvecs), tuple(rvecs)))
        pltpu.sync_copy(wrowss[b], out_hbm.at[pl.ds(base, CHUNK)])

    prep_idx(0, 0)
    start_gather(0)
    prep_idx(1, 1)
    start_gather(1)
    comb_cp.wait()

    def outer(oi, _):
        for b in range(2):
            ci = oi * 2 + b
            finish_chunk(ci, b)

            @pl.when(ci + 2 < NCHUNK)
            def _():
                prep_idx(ci + 2, b)
                start_gather(b)
        return 0

    lax.fori_loop(0, NCHUNK // 2, outer, 0)


def kernel(input_ids, token_type_ids, word_emb, pos_emb, tok_emb):
    combined = _build_combined(pos_emb, tok_emb).reshape(-1)
    ids_flat = input_ids.reshape(-1)
    tt_flat = token_type_ids.reshape(-1)
    out = _sc_embed(word_emb, combined, ids_flat, tt_flat)
    return out.reshape(input_ids.shape[0], input_ids.shape[1], HID)


# speedup vs baseline: 3.4320x; 3.4320x over previous
import functools

import jax
import jax.numpy as jnp
from jax import lax
from jax.experimental import pallas as pl
from jax.experimental.pallas import tpu as pltpu
from jax.experimental.pallas import tpu_sc as plsc

VOCAB = 100000
HID = 128
CTX = 200
NROW = 1024 * 200
NC = 2
NS = 16
NW = NC * NS
RPW = NROW // NW
CHUNK = 128
NCHUNK = RPW // CHUNK
LANES = 16
NBUF = 3


def _combine_body(pos_ref, tok_ref, out_ref):
    out_ref[0:CTX, :] = pos_ref[...] + tok_ref[0:1, :]
    out_ref[CTX:2 * CTX, :] = pos_ref[...] + tok_ref[1:2, :]


def _build_combined(pos_emb, tok_emb):
    return pl.pallas_call(
        _combine_body,
        out_shape=jax.ShapeDtypeStruct((2 * CTX, HID), jnp.float32),
    )(pos_emb, tok_emb)


_sc_mesh = plsc.VectorSubcoreMesh(core_axis_name="c", subcore_axis_name="s")


@functools.partial(
    pl.kernel,
    out_type=jax.ShapeDtypeStruct((NROW, HID), jnp.float32),
    mesh=_sc_mesh,
    scratch_types=[
        pltpu.VMEM((2 * CTX, HID), jnp.float32),
        pltpu.VMEM((NBUF, CHUNK), jnp.int32),
        pltpu.VMEM((NBUF, CHUNK), jnp.int32),
        pltpu.VMEM((NBUF, CHUNK), jnp.int32),
        pltpu.VMEM((NBUF, CHUNK, HID), jnp.float32),
        pltpu.SemaphoreType.DMA,
        pltpu.SemaphoreType.DMA,
        pltpu.SemaphoreType.DMA,
        pltpu.SemaphoreType.DMA,
        pltpu.SemaphoreType.DMA,
        pltpu.SemaphoreType.DMA,
        pltpu.SemaphoreType.DMA,
    ],
    compiler_params=pltpu.CompilerParams(needs_layout_passes=False),
)
def _sc_embed(word_hbm, comb_hbm, ids_hbm, tt_hbm, out_hbm,
              comb_v, widx, ttv, cidx, wrows,
              csem, gsem0, gsem1, gsem2, osem0, osem1, osem2):
    wid = lax.axis_index("s") * NC + lax.axis_index("c")
    row0 = wid * RPW
    gsems = (gsem0, gsem1, gsem2)
    osems = (osem0, osem1, osem2)

    comb_cp = pltpu.async_copy(comb_hbm, comb_v, csem)

    def prep_idx(ci, b):
        base = row0 + ci * CHUNK
        pltpu.sync_copy(ids_hbm.at[pl.ds(base, CHUNK)], widx.at[b])
        pltpu.sync_copy(tt_hbm.at[pl.ds(base, CHUNK)], ttv.at[b])

        def idx_body(j, _):
            o = j * LANES
            n = base + o + lax.iota(jnp.int32, LANES)
            s = n % CTX
            cidx[b, pl.ds(o, LANES)] = ttv[b, pl.ds(o, LANES)] * CTX + s
            return 0

        lax.fori_loop(0, CHUNK // LANES, idx_body, 0)

    def start_gather(b):
        pltpu.async_copy(word_hbm.at[widx.at[b]], wrows.at[b], gsems[b])

    def wait_gather(b):
        pltpu.make_async_copy(word_hbm.at[widx.at[b]], wrows.at[b],
                              gsems[b]).wait()

    def start_out(ci, b):
        base = row0 + ci * CHUNK
        pltpu.async_copy(wrows.at[b], out_hbm.at[pl.ds(base, CHUNK)],
                         osems[b])

    def wait_out(b):
        pltpu.make_async_copy(wrows.at[b], out_hbm.at[pl.ds(row0, CHUNK)],
                              osems[b]).wait()

    def add_chunk(b):
        def group_body(g, _):
            o = g * LANES
            cvec = cidx[b, pl.ds(o, LANES)]
            for i in range(LANES):
                cr = cvec[i]
                r = o + i
                for j in range(HID // LANES):
                    sl = pl.ds(j * LANES, LANES)
                    wrows[b, r, sl] = wrows[b, r, sl] + comb_v[cr, sl]
            return 0

        lax.fori_loop(0, CHUNK // LANES, group_body, 0)

    prep_idx(0, 0)
    start_gather(0)
    prep_idx(1, 1)
    start_gather(1)
    comb_cp.wait()

    def outer(oi, _):
        for k in range(NBUF):
            ci = oi * NBUF + k

            @pl.when(ci < NCHUNK)
            def _():
                b = k
                wait_gather(b)
                add_chunk(b)
                start_out(ci, b)

                @pl.when(ci + 2 < NCHUNK)
                def _():
                    pb = (k + 2) % NBUF

                    @pl.when(ci >= 1)
                    def _():
                        wait_out(pb)

                    prep_idx(ci + 2, pb)
                    start_gather(pb)
        return 0

    lax.fori_loop(0, (NCHUNK + NBUF - 1) // NBUF, outer, 0)

    wait_out(0)
    wait_out(1)
    wait_out(2)


def kernel(input_ids, token_type_ids, word_emb, pos_emb, tok_emb):
    combined = _build_combined(pos_emb, tok_emb)
    ids_flat = input_ids.reshape(-1)
    tt_flat = token_type_ids.reshape(-1)
    out = _sc_embed(word_emb, combined, ids_flat, tt_flat)
    return out.reshape(input_ids.shape[0], input_ids.shape[1], HID)


# speedup vs baseline: 5.8417x; 1.7021x over previous
import functools

import jax
import jax.numpy as jnp
from jax import lax
from jax.experimental import pallas as pl
from jax.experimental.pallas import tpu as pltpu
from jax.experimental.pallas import tpu_sc as plsc

VOCAB = 100000
HID = 128
CTX = 200
NROW = 1024 * 200
NC = 2
NS = 16
NW = NC * NS
RPW = NROW // NW
CHUNK = 128
NCHUNK = RPW // CHUNK
LANES = 16
NBUF = 3


def _combine_body(pos_ref, tok_ref, out_ref):
    out_ref[0:CTX, :] = pos_ref[...] + tok_ref[0:1, :]
    out_ref[CTX:2 * CTX, :] = pos_ref[...] + tok_ref[1:2, :]


def _build_combined(pos_emb, tok_emb):
    return pl.pallas_call(
        _combine_body,
        out_shape=jax.ShapeDtypeStruct((2 * CTX, HID), jnp.float32),
    )(pos_emb, tok_emb)


_sc_mesh = plsc.VectorSubcoreMesh(core_axis_name="c", subcore_axis_name="s")


@functools.partial(
    pl.kernel,
    out_type=jax.ShapeDtypeStruct((NROW, HID), jnp.float32),
    mesh=_sc_mesh,
    scratch_types=[
        pltpu.VMEM((NBUF, CHUNK), jnp.int32),
        pltpu.VMEM((NBUF, CHUNK), jnp.int32),
        pltpu.VMEM((NBUF, CHUNK), jnp.int32),
        pltpu.VMEM((NBUF, CHUNK, HID), jnp.float32),
        pltpu.VMEM((NBUF, CHUNK, HID), jnp.float32),
        pltpu.SemaphoreType.DMA,
        pltpu.SemaphoreType.DMA,
        pltpu.SemaphoreType.DMA,
        pltpu.SemaphoreType.DMA,
        pltpu.SemaphoreType.DMA,
        pltpu.SemaphoreType.DMA,
    ],
)
def _sc_embed(word_hbm, comb_hbm, ids_hbm, tt_hbm, out_hbm,
              widx, ttv, cidx, wrows, crows,
              gsem0, gsem1, gsem2, osem0, osem1, osem2):
    wid = lax.axis_index("s") * NC + lax.axis_index("c")
    row0 = wid * RPW
    gsems = (gsem0, gsem1, gsem2)
    osems = (osem0, osem1, osem2)

    def prep_idx(ci, b):
        base = row0 + ci * CHUNK
        pltpu.sync_copy(ids_hbm.at[pl.ds(base, CHUNK)], widx.at[b])
        pltpu.sync_copy(tt_hbm.at[pl.ds(base, CHUNK)], ttv.at[b])

        def idx_body(j, _):
            o = j * LANES
            n = base + o + lax.iota(jnp.int32, LANES)
            s = n % CTX
            cidx[b, pl.ds(o, LANES)] = ttv[b, pl.ds(o, LANES)] * CTX + s
            return 0

        lax.fori_loop(0, CHUNK // LANES, idx_body, 0)

    def start_gather(b):
        pltpu.async_copy(word_hbm.at[widx.at[b]], wrows.at[b], gsems[b])
        pltpu.async_copy(comb_hbm.at[cidx.at[b]], crows.at[b], gsems[b])

    def wait_gather(b):
        pltpu.make_async_copy(word_hbm.at[widx.at[b]], wrows.at[b],
                              gsems[b]).wait()
        pltpu.make_async_copy(comb_hbm.at[cidx.at[b]], crows.at[b],
                              gsems[b]).wait()

    def start_out(ci, b):
        base = row0 + ci * CHUNK
        pltpu.async_copy(wrows.at[b], out_hbm.at[pl.ds(base, CHUNK)],
                         osems[b])

    def wait_out(b):
        pltpu.make_async_copy(wrows.at[b], out_hbm.at[pl.ds(row0, CHUNK)],
                              osems[b]).wait()

    def add_chunk(b):
        def add_body(r, _):
            for j in range(HID // LANES):
                sl = pl.ds(j * LANES, LANES)
                wrows[b, r, sl] = wrows[b, r, sl] + crows[b, r, sl]
            return 0

        lax.fori_loop(0, CHUNK, add_body, 0)

    prep_idx(0, 0)
    start_gather(0)
    prep_idx(1, 1)
    start_gather(1)

    def outer(oi, _):
        for k in range(NBUF):
            ci = oi * NBUF + k

            @pl.when(ci < NCHUNK)
            def _():
                b = k
                wait_gather(b)
                add_chunk(b)
                start_out(ci, b)

                @pl.when(ci + 2 < NCHUNK)
                def _():
                    pb = (k + 2) % NBUF

                    @pl.when(ci >= 1)
                    def _():
                        wait_out(pb)

                    prep_idx(ci + 2, pb)
                    start_gather(pb)
        return 0

    lax.fori_loop(0, (NCHUNK + NBUF - 1) // NBUF, outer, 0)

    wait_out(0)
    wait_out(1)
    wait_out(2)


def kernel(input_ids, token_type_ids, word_emb, pos_emb, tok_emb):
    combined = _build_combined(pos_emb, tok_emb)
    ids_flat = input_ids.reshape(-1)
    tt_flat = token_type_ids.reshape(-1)
    out = _sc_embed(word_emb, combined, ids_flat, tt_flat)
    return out.reshape(input_ids.shape[0], input_ids.shape[1], HID)


# speedup vs baseline: 6.3260x; 1.0829x over previous
import functools

import jax
import jax.numpy as jnp
from jax import lax
from jax.experimental import pallas as pl
from jax.experimental.pallas import tpu as pltpu
from jax.experimental.pallas import tpu_sc as plsc

VOCAB = 100000
HID = 128
CTX = 200
NROW = 1024 * 200
NC = 2
NS = 16
NW = NC * NS
RPW = NROW // NW
CHUNK = 128
NCHUNK = RPW // CHUNK
LANES = 16
NBUF = 3
CTXP = 256
BROWS = 2 * CTXP // NS

_sc_mesh = plsc.VectorSubcoreMesh(core_axis_name="c", subcore_axis_name="s")


@functools.partial(
    pl.kernel,
    out_type=(
        jax.ShapeDtypeStruct((NROW, HID), jnp.float32),
        jax.ShapeDtypeStruct((NC * 2 * CTXP, HID), jnp.float32),
    ),
    mesh=_sc_mesh,
    scratch_types=[
        pltpu.VMEM((NBUF, CHUNK), jnp.int32),
        pltpu.VMEM((NBUF, CHUNK), jnp.int32),
        pltpu.VMEM((NBUF, CHUNK), jnp.int32),
        pltpu.VMEM((NBUF, CHUNK, HID), jnp.float32),
        pltpu.VMEM((NBUF, CHUNK, HID), jnp.float32),
        pltpu.SemaphoreType.DMA,
        pltpu.SemaphoreType.DMA,
        pltpu.SemaphoreType.DMA,
        pltpu.SemaphoreType.DMA,
        pltpu.SemaphoreType.DMA,
        pltpu.SemaphoreType.DMA,
    ],
)
def _sc_embed(word_hbm, pos_hbm, tok_hbm, ids_hbm, tt_hbm,
              out_hbm, comb_hbm,
              widx, ttv, cidx, wrows, crows,
              gsem0, gsem1, gsem2, osem0, osem1, osem2):
    cid = lax.axis_index("c")
    sid = lax.axis_index("s")
    wid = sid * NC + cid
    row0 = wid * RPW
    gsems = (gsem0, gsem1, gsem2)
    osems = (osem0, osem1, osem2)

    t = sid // (NS // 2)
    s0 = (sid * BROWS) % CTXP
    pltpu.sync_copy(pos_hbm.at[pl.ds(s0, BROWS)],
                    crows.at[0].at[pl.ds(0, BROWS)])
    pltpu.sync_copy(tok_hbm, crows.at[0].at[pl.ds(40, 2)])
    for r in range(BROWS):
        for j in range(HID // LANES):
            sl = pl.ds(j * LANES, LANES)
            crows[0, r, sl] = crows[0, r, sl] + crows[0, 40 + t, sl]
    pltpu.sync_copy(crows.at[0].at[pl.ds(0, BROWS)],
                    comb_hbm.at[pl.ds(cid * 2 * CTXP + sid * BROWS, BROWS)])
    plsc.subcore_barrier()

    cbase = cid * 2 * CTXP

    def prep_idx(ci, b):
        base = row0 + ci * CHUNK
        pltpu.sync_copy(ids_hbm.at[pl.ds(base, CHUNK)], widx.at[b])
        pltpu.sync_copy(tt_hbm.at[pl.ds(base, CHUNK)], ttv.at[b])

        def idx_body(j, _):
            o = j * LANES
            n = base + o + lax.iota(jnp.int32, LANES)
            s = n % CTX
            cidx[b, pl.ds(o, LANES)] = (cbase
                                        + ttv[b, pl.ds(o, LANES)] * CTXP + s)
            return 0

        lax.fori_loop(0, CHUNK // LANES, idx_body, 0)

    def start_gather(b):
        pltpu.async_copy(word_hbm.at[widx.at[b]], wrows.at[b], gsems[b])
        pltpu.async_copy(comb_hbm.at[cidx.at[b]], crows.at[b], gsems[b])

    def wait_gather(b):
        pltpu.make_async_copy(word_hbm.at[widx.at[b]], wrows.at[b],
                              gsems[b]).wait()
        pltpu.make_async_copy(comb_hbm.at[cidx.at[b]], crows.at[b],
                              gsems[b]).wait()

    def start_out(ci, b):
        base = row0 + ci * CHUNK
        pltpu.async_copy(wrows.at[b], out_hbm.at[pl.ds(base, CHUNK)],
                         osems[b])

    def wait_out(b):
        pltpu.make_async_copy(wrows.at[b], out_hbm.at[pl.ds(row0, CHUNK)],
                              osems[b]).wait()

    def add_chunk(b):
        def add_body(r, _):
            for j in range(HID // LANES):
                sl = pl.ds(j * LANES, LANES)
                wrows[b, r, sl] = wrows[b, r, sl] + crows[b, r, sl]
            return 0

        lax.fori_loop(0, CHUNK, add_body, 0)

    prep_idx(0, 0)
    start_gather(0)
    prep_idx(1, 1)
    start_gather(1)

    def outer(oi, _):
        for k in range(NBUF):
            ci = oi * NBUF + k

            @pl.when(ci < NCHUNK)
            def _():
                b = k
                wait_gather(b)
                add_chunk(b)
                start_out(ci, b)

                @pl.when(ci + 2 < NCHUNK)
                def _():
                    pb = (k + 2) % NBUF

                    @pl.when(ci >= 1)
                    def _():
                        wait_out(pb)

                    prep_idx(ci + 2, pb)
                    start_gather(pb)
        return 0

    lax.fori_loop(0, (NCHUNK + NBUF - 1) // NBUF, outer, 0)

    wait_out(0)
    wait_out(1)
    wait_out(2)


def kernel(input_ids, token_type_ids, word_emb, pos_emb, tok_emb):
    pos_pad = jnp.zeros((CTXP, HID), jnp.float32).at[:CTX].set(pos_emb)
    ids_flat = input_ids.reshape(-1)
    tt_flat = token_type_ids.reshape(-1)
    out, _ = _sc_embed(word_emb, pos_pad, tok_emb, ids_flat, tt_flat)
    return out.reshape(input_ids.shape[0], input_ids.shape[1], HID)
